# Initial kernel scaffold; baseline (speedup 1.0000x reference)
import jax
import jax.numpy as jnp
from jax.experimental import pallas as pl


def kernel(micro_dynamic, micro_edge_index, micro_node_degrees, com, com_edges, com_weights, macro_dynamic, macro_static, k_in, gcn_W, gcn_b, macro_gcn_W, macro_gcn_b, attn_W, attn_b, lstm_Wih, lstm_Whh, lstm_bih, lstm_bhh, mlstm_Wih, mlstm_Whh, mlstm_bih, mlstm_bhh, gout_W, gout_b):
    raise NotImplementedError("write your pallas kernel here")


# speedup vs baseline: 7.0138x; 7.0138x over previous
import jax
import jax.numpy as jnp
from jax.experimental import pallas as pl

HID = 128
DAYS = 10
NLANE = 128
TU = 1000


def _pad2(x, r, c):
    return jnp.zeros((r, c), jnp.float32).at[: x.shape[0], : x.shape[1]].set(x)


def _cstage(gc_ref, gm_ref, md_ref, nd_ref, wa2z_ref, wa2y_ref, ab_ref,
            a1_ref, a2_ref, whh_ref, mb_ref, sel_ref, zyw_ref, qtk_ref):
    gc = gc_ref[...]
    gm = gm_ref[...]
    wa2z = wa2z_ref[...]
    wa2y = wa2y_ref[...]
    ab = ab_ref[...]
    a1 = a1_ref[...]
    a2 = a2_ref[...]
    whh = whh_ref[...]
    mb = mb_ref[...]
    h = jnp.zeros((NLANE, NLANE), jnp.float32)
    c = jnp.zeros((NLANE, NLANE), jnp.float32)
    f32 = jnp.float32
    for d in range(DAYS):
        zt = jnp.dot(gc, md_ref[d], preferred_element_type=f32)
        yt = jnp.dot(gm, nd_ref[d], preferred_element_type=f32)
        zyw_ref[d] = (jnp.dot(zt, wa2z, preferred_element_type=f32)
                      + jnp.dot(yt, wa2y, preferred_element_type=f32) + ab)
        qz = jax.nn.relu(zt)
        qy = jax.nn.relu(yt)
        g = (jnp.dot(qz, a1, preferred_element_type=f32)
             + jnp.dot(qy, a2, preferred_element_type=f32)
             + jnp.dot(h, whh, preferred_element_type=f32) + mb)
        gi = jax.nn.sigmoid(jnp.dot(g, sel_ref[0], preferred_element_type=f32))
        gf = jax.nn.sigmoid(jnp.dot(g, sel_ref[1], preferred_element_type=f32))
        gg = jnp.tanh(jnp.dot(g, sel_ref[2], preferred_element_type=f32))
        go = jax.nn.sigmoid(jnp.dot(g, sel_ref[3], preferred_element_type=f32))
        c = gf * c + gi * gg
        h = go * jnp.tanh(c)
    qtk_ref[...] = h


def _nodestage(gx_ref, oh_ref, kt_ref, md_ref, zyw_ref, wa1_ref, wk_ref,
               ab_ref, wih_ref, whh_ref, lb_ref, ys_ref):
    f32 = jnp.float32
    gx = gx_ref[...]
    oh = oh_ref[...]
    wa1 = wa1_ref[...]
    wih = wih_ref[...]
    whh = whh_ref[...]
    lb = lb_ref[...]
    kterm = jnp.dot(kt_ref[...], wk_ref[...], preferred_element_type=f32) + ab_ref[...]
    h = jnp.zeros((TU, HID), jnp.float32)
    c = jnp.zeros((TU, HID), jnp.float32)
    for d in range(DAYS):
        ht = jnp.dot(gx, md_ref[d], preferred_element_type=f32)
        hr = jax.nn.relu(ht)
        hv1 = jax.nn.relu(jnp.dot(hr, wa1, preferred_element_type=f32)
                          + jnp.dot(oh, zyw_ref[d], preferred_element_type=f32)) * hr
        hv2 = jax.nn.relu(jnp.dot(hv1, wa1, preferred_element_type=f32) + kterm) * hv1
        g = (jnp.dot(hv2, wih, preferred_element_type=f32)
             + jnp.dot(h, whh, preferred_element_type=f32) + lb)
        gi = jax.nn.sigmoid(g[:, 0:HID])
        gf = jax.nn.sigmoid(g[:, HID:2 * HID])
        gg = jnp.tanh(g[:, 2 * HID:3 * HID])
        go = jax.nn.sigmoid(g[:, 3 * HID:4 * HID])
        c = gf * c + gi * gg
        h = go * jnp.tanh(c)
        ys_ref[d] = jax.nn.relu(h)


def _headmm(x_ref, w_ref, o_ref):
    o_ref[...] = jnp.dot(x_ref[...], w_ref[...], preferred_element_type=jnp.float32)


def _softmax(x_ref, b_ref, o_ref):
    y = x_ref[...] + b_ref[...]
    m = jnp.max(y, axis=1, keepdims=True)
    e = jnp.exp(y - m)
    o_ref[...] = e / jnp.sum(e, axis=1, keepdims=True)


def kernel(micro_dynamic, micro_edge_index, micro_node_degrees, com, com_edges,
           com_weights, macro_dynamic, macro_static, k_in, gcn_W, gcn_b,
           macro_gcn_W, macro_gcn_b, attn_W, attn_b, lstm_Wih, lstm_Whh,
           lstm_bih, lstm_bhh, mlstm_Wih, mlstm_Whh, mlstm_bih, mlstm_bhh,
           gout_W, gout_b):
    f32 = jnp.float32
    U = micro_dynamic.shape[0]
    C = macro_dynamic.shape[0]
    OUT = gout_W.shape[1]
    src, dst = micro_edge_index[0], micro_edge_index[1]

    deg = jax.ops.segment_sum(jnp.ones_like(src, f32), dst, num_segments=U) + 1.0
    dinv = jnp.where(deg > 0, deg ** -0.5, 0.0)
    norm = dinv[src] * dinv[dst]
    slf = dinv * dinv
    bx = micro_dynamic[:, 0, :]
    xall = jnp.concatenate([bx, micro_node_degrees[:, None]], 1)
    G = jax.ops.segment_sum(norm[:, None] * xall[src], dst, num_segments=U) \
        + slf[:, None] * xall

    counts = jnp.maximum(jnp.bincount(com, length=C), 1).astype(f32)
    Gc = jax.ops.segment_sum(G, com, num_segments=C) / counts[:, None]

    msrc, mdst = com_edges[0], com_edges[1]
    degm = jax.ops.segment_sum(com_weights, mdst, num_segments=C) + 1.0
    dinvm = jnp.where(degm > 0, degm ** -0.5, 0.0)
    normm = dinvm[msrc] * com_weights * dinvm[mdst]
    slfm = dinvm * dinvm
    mx = jnp.concatenate([macro_dynamic[:, 0, :, 0], macro_dynamic[:, 0, :, 1],
                          macro_static[:, None]], 1)
    Gm = jax.ops.segment_sum(normm[:, None] * mx[msrc], mdst, num_segments=C) \
        + slfm[:, None] * mx

    eye = jnp.eye(NLANE, dtype=f32)
    md = jnp.stack([jnp.outer(eye[d], gcn_W[0]) + jnp.outer(eye[10], gcn_W[1])
                    + jnp.outer(eye[24], gcn_b) for d in range(DAYS)])
    nd = jnp.stack([jnp.outer(eye[d], macro_gcn_W[0])
                    + jnp.outer(eye[10 + d], macro_gcn_W[1])
                    + jnp.outer(eye[20], macro_gcn_W[2])
                    + jnp.outer(eye[24], macro_gcn_b) for d in range(DAYS)])
    wa1 = attn_W[:HID]
    wa2z = attn_W[HID:2 * HID]
    wa2y = attn_W[2 * HID:3 * HID]
    wk = attn_W[HID:].reshape(32, 8, HID).sum(0)
    ab = attn_b[None, :]
    mwih_t = mlstm_Wih.T
    a1 = _pad2(mwih_t[:HID], NLANE, NLANE)
    a2 = _pad2(mwih_t[HID:], NLANE, NLANE)
    mwhh = _pad2(mlstm_Whh.T, NLANE, NLANE)
    mb = _pad2((mlstm_bih + mlstm_bhh)[None, :], 1, NLANE)
    nh = mlstm_Whh.shape[0] // 4
    sel = jnp.zeros((4, NLANE, NLANE), f32)
    for j in range(4):
        for cix in range(nh):
            sel = sel.at[j, j * nh + cix, cix].set(1.0)
    wih = lstm_Wih.T
    whh = lstm_Whh.T
    lb = (lstm_bih + lstm_bhh)[None, :]

    gc_p = _pad2(Gc, NLANE, NLANE).at[:, 24].set(1.0)
    gm_p = _pad2(Gm, NLANE, NLANE).at[:, 24].set(1.0)

    zyw, qtk = pl.pallas_call(
        _cstage,
        out_shape=[jax.ShapeDtypeStruct((DAYS, NLANE, NLANE), f32),
                   jax.ShapeDtypeStruct((NLANE, NLANE), f32)],
    )(gc_p, gm_p, md, nd, wa2z, wa2y, ab, a1, a2, mwhh, mb, sel)

    gx = _pad2(G, U, NLANE).at[:, 24].set(1.0)
    onehot = (com[:, None] == jnp.arange(NLANE, dtype=com.dtype)[None, :]
              ).astype(f32)
    ktp = k_in.T

    grid = U // TU
    ys = pl.pallas_call(
        _nodestage,
        grid=(grid,),
        in_specs=[
            pl.BlockSpec((TU, NLANE), lambda i: (i, 0)),
            pl.BlockSpec((TU, NLANE), lambda i: (i, 0)),
            pl.BlockSpec((TU, 8), lambda i: (i, 0)),
            pl.BlockSpec((DAYS, NLANE, NLANE), lambda i: (0, 0, 0)),
            pl.BlockSpec((DAYS, NLANE, NLANE), lambda i: (0, 0, 0)),
            pl.BlockSpec((HID, HID), lambda i: (0, 0)),
            pl.BlockSpec((8, HID), lambda i: (0, 0)),
            pl.BlockSpec((1, HID), lambda i: (0, 0)),
            pl.BlockSpec((HID, 4 * HID), lambda i: (0, 0)),
            pl.BlockSpec((HID, 4 * HID), lambda i: (0, 0)),
            pl.BlockSpec((1, 4 * HID), lambda i: (0, 0)),
        ],
        out_specs=pl.BlockSpec((DAYS, TU, HID), lambda i: (0, i, 0)),
        out_shape=jax.ShapeDtypeStruct((DAYS, U, HID), f32),
    )(gx, onehot, ktp, md, zyw, wa1, wk, ab, wih, whh, lb)

    hflat = ys.reshape(U, DAYS * HID)
    gw = _pad2(gout_W, DAYS * HID, NLANE)
    xw = pl.pallas_call(
        _headmm,
        grid=(grid,),
        in_specs=[pl.BlockSpec((TU, DAYS * HID), lambda i: (i, 0)),
                  pl.BlockSpec((DAYS * HID, NLANE), lambda i: (0, 0))],
        out_specs=pl.BlockSpec((TU, NLANE), lambda i: (i, 0)),
        out_shape=jax.ShapeDtypeStruct((U, NLANE), f32),
    )(hflat, gw)[:, :OUT]

    g2 = jax.ops.segment_sum(norm[:, None] * xw[src], dst, num_segments=U) \
        + slf[:, None] * xw

    bvec = jnp.full((1, NLANE), -1e30, f32).at[0, :OUT].set(gout_b)
    g2p = _pad2(g2, U, NLANE)
    probs = pl.pallas_call(
        _softmax,
        grid=(grid,),
        in_specs=[pl.BlockSpec((TU, NLANE), lambda i: (i, 0)),
                  pl.BlockSpec((1, NLANE), lambda i: (0, 0))],
        out_specs=pl.BlockSpec((TU, NLANE), lambda i: (i, 0)),
        out_shape=jax.ShapeDtypeStruct((U, NLANE), f32),
    )(g2p, bvec)[:, :OUT]

    micro = probs[None]
    macro = qtk[:C, :nh][None]
    return (micro, macro)


# speedup vs baseline: 14.3241x; 2.0423x over previous
import jax
import jax.numpy as jnp
from jax.experimental import pallas as pl

HID = 128
DAYS = 10
NLANE = 128
TU = 1000


def _pad2(x, r, c):
    return jnp.zeros((r, c), jnp.float32).at[: x.shape[0], : x.shape[1]].set(x)


def _cstage(gc_ref, gm_ref, md_ref, nd_ref, wa2z_ref, wa2y_ref, ab_ref,
            a1_ref, a2_ref, whh_ref, mb_ref, sel_ref, zyw_ref, qtk_ref):
    gc = gc_ref[...]
    gm = gm_ref[...]
    wa2z = wa2z_ref[...]
    wa2y = wa2y_ref[...]
    ab = ab_ref[...]
    a1 = a1_ref[...]
    a2 = a2_ref[...]
    whh = whh_ref[...]
    mb = mb_ref[...]
    h = jnp.zeros((NLANE, NLANE), jnp.float32)
    c = jnp.zeros((NLANE, NLANE), jnp.float32)
    f32 = jnp.float32
    for d in range(DAYS):
        zt = jnp.dot(gc, md_ref[d], preferred_element_type=f32)
        yt = jnp.dot(gm, nd_ref[d], preferred_element_type=f32)
        zyw_ref[d] = (jnp.dot(zt, wa2z, preferred_element_type=f32)
                      + jnp.dot(yt, wa2y, preferred_element_type=f32) + ab)
        qz = jax.nn.relu(zt)
        qy = jax.nn.relu(yt)
        g = (jnp.dot(qz, a1, preferred_element_type=f32)
             + jnp.dot(qy, a2, preferred_element_type=f32)
             + jnp.dot(h, whh, preferred_element_type=f32) + mb)
        gi = jax.nn.sigmoid(jnp.dot(g, sel_ref[0], preferred_element_type=f32))
        gf = jax.nn.sigmoid(jnp.dot(g, sel_ref[1], preferred_element_type=f32))
        gg = jnp.tanh(jnp.dot(g, sel_ref[2], preferred_element_type=f32))
        go = jax.nn.sigmoid(jnp.dot(g, sel_ref[3], preferred_element_type=f32))
        c = gf * c + gi * gg
        h = go * jnp.tanh(c)
    qtk_ref[...] = h


def _nodestage(gx_ref, oh_ref, kt_ref, md_ref, zyw_ref, wa1_ref, wk_ref,
               ab_ref, wih_ref, whh_ref, lb_ref, ys_ref):
    f32 = jnp.float32
    gx = gx_ref[...]
    oh = oh_ref[...]
    wa1 = wa1_ref[...]
    wih = wih_ref[...]
    whh = whh_ref[...]
    lb = lb_ref[...]
    kterm = jnp.dot(kt_ref[...], wk_ref[...], preferred_element_type=f32) + ab_ref[...]
    h = jnp.zeros((TU, HID), jnp.float32)
    c = jnp.zeros((TU, HID), jnp.float32)
    for d in range(DAYS):
        ht = jnp.dot(gx, md_ref[d], preferred_element_type=f32)
        hr = jax.nn.relu(ht)
        hv1 = jax.nn.relu(jnp.dot(hr, wa1, preferred_element_type=f32)
                          + jnp.dot(oh, zyw_ref[d], preferred_element_type=f32)) * hr
        hv2 = jax.nn.relu(jnp.dot(hv1, wa1, preferred_element_type=f32) + kterm) * hv1
        g = (jnp.dot(hv2, wih, preferred_element_type=f32)
             + jnp.dot(h, whh, preferred_element_type=f32) + lb)
        gi = jax.nn.sigmoid(g[:, 0:HID])
        gf = jax.nn.sigmoid(g[:, HID:2 * HID])
        gg = jnp.tanh(g[:, 2 * HID:3 * HID])
        go = jax.nn.sigmoid(g[:, 3 * HID:4 * HID])
        c = gf * c + gi * gg
        h = go * jnp.tanh(c)
        ys_ref[d] = jax.nn.relu(h)


def _headmm(x_ref, w_ref, o_ref):
    o_ref[...] = jnp.dot(x_ref[...], w_ref[...], preferred_element_type=jnp.float32)


def _softmax(x_ref, b_ref, o_ref):
    y = x_ref[...] + b_ref[...]
    m = jnp.max(y, axis=1, keepdims=True)
    e = jnp.exp(y - m)
    o_ref[...] = e / jnp.sum(e, axis=1, keepdims=True)


def kernel(micro_dynamic, micro_edge_index, micro_node_degrees, com, com_edges,
           com_weights, macro_dynamic, macro_static, k_in, gcn_W, gcn_b,
           macro_gcn_W, macro_gcn_b, attn_W, attn_b, lstm_Wih, lstm_Whh,
           lstm_bih, lstm_bhh, mlstm_Wih, mlstm_Whh, mlstm_bih, mlstm_bhh,
           gout_W, gout_b):
    f32 = jnp.float32
    U = micro_dynamic.shape[0]
    C = macro_dynamic.shape[0]
    OUT = gout_W.shape[1]
    src, dst = micro_edge_index[0], micro_edge_index[1]

    deg = jax.ops.segment_sum(jnp.ones_like(src, f32), dst, num_segments=U) + 1.0
    dinv = jnp.where(deg > 0, deg ** -0.5, 0.0)
    slf = dinv * dinv
    bx = micro_dynamic[:, 0, :]
    xall = jnp.concatenate([bx, micro_node_degrees[:, None]], 1)
    xs = dinv[:, None] * xall
    G = dinv[:, None] * jax.ops.segment_sum(xs[src], dst, num_segments=U) \
        + slf[:, None] * xall

    counts = jnp.maximum(jnp.bincount(com, length=C), 1).astype(f32)
    Gc = jax.ops.segment_sum(G, com, num_segments=C) / counts[:, None]

    msrc, mdst = com_edges[0], com_edges[1]
    degm = jax.ops.segment_sum(com_weights, mdst, num_segments=C) + 1.0
    dinvm = jnp.where(degm > 0, degm ** -0.5, 0.0)
    normm = dinvm[msrc] * com_weights * dinvm[mdst]
    slfm = dinvm * dinvm
    mx = jnp.concatenate([macro_dynamic[:, 0, :, 0], macro_dynamic[:, 0, :, 1],
                          macro_static[:, None]], 1)
    Gm = jax.ops.segment_sum(normm[:, None] * mx[msrc], mdst, num_segments=C) \
        + slfm[:, None] * mx

    eye = jnp.eye(NLANE, dtype=f32)
    md = jnp.stack([jnp.outer(eye[d], gcn_W[0]) + jnp.outer(eye[10], gcn_W[1])
                    + jnp.outer(eye[24], gcn_b) for d in range(DAYS)])
    nd = jnp.stack([jnp.outer(eye[d], macro_gcn_W[0])
                    + jnp.outer(eye[10 + d], macro_gcn_W[1])
                    + jnp.outer(eye[20], macro_gcn_W[2])
                    + jnp.outer(eye[24], macro_gcn_b) for d in range(DAYS)])
    wa1 = attn_W[:HID]
    wa2z = attn_W[HID:2 * HID]
    wa2y = attn_W[2 * HID:3 * HID]
    wk = attn_W[HID:].reshape(32, 8, HID).sum(0)
    ab = attn_b[None, :]
    mwih_t = mlstm_Wih.T
    a1 = _pad2(mwih_t[:HID], NLANE, NLANE)
    a2 = _pad2(mwih_t[HID:], NLANE, NLANE)
    mwhh = _pad2(mlstm_Whh.T, NLANE, NLANE)
    mb = _pad2((mlstm_bih + mlstm_bhh)[None, :], 1, NLANE)
    nh = mlstm_Whh.shape[0] // 4
    sel = jnp.zeros((4, NLANE, NLANE), f32)
    for j in range(4):
        for cix in range(nh):
            sel = sel.at[j, j * nh + cix, cix].set(1.0)
    wih = lstm_Wih.T
    whh = lstm_Whh.T
    lb = (lstm_bih + lstm_bhh)[None, :]

    gc_p = _pad2(Gc, NLANE, NLANE).at[:, 24].set(1.0)
    gm_p = _pad2(Gm, NLANE, NLANE).at[:, 24].set(1.0)

    zyw, qtk = pl.pallas_call(
        _cstage,
        out_shape=[jax.ShapeDtypeStruct((DAYS, NLANE, NLANE), f32),
                   jax.ShapeDtypeStruct((NLANE, NLANE), f32)],
    )(gc_p, gm_p, md, nd, wa2z, wa2y, ab, a1, a2, mwhh, mb, sel)

    gx = _pad2(G, U, NLANE).at[:, 24].set(1.0)
    onehot = (com[:, None] == jnp.arange(NLANE, dtype=com.dtype)[None, :]
              ).astype(f32)
    ktp = k_in.T

    grid = U // TU
    ys = pl.pallas_call(
        _nodestage,
        grid=(grid,),
        in_specs=[
            pl.BlockSpec((TU, NLANE), lambda i: (i, 0)),
            pl.BlockSpec((TU, NLANE), lambda i: (i, 0)),
            pl.BlockSpec((TU, 8), lambda i: (i, 0)),
            pl.BlockSpec((DAYS, NLANE, NLANE), lambda i: (0, 0, 0)),
            pl.BlockSpec((DAYS, NLANE, NLANE), lambda i: (0, 0, 0)),
            pl.BlockSpec((HID, HID), lambda i: (0, 0)),
            pl.BlockSpec((8, HID), lambda i: (0, 0)),
            pl.BlockSpec((1, HID), lambda i: (0, 0)),
            pl.BlockSpec((HID, 4 * HID), lambda i: (0, 0)),
            pl.BlockSpec((HID, 4 * HID), lambda i: (0, 0)),
            pl.BlockSpec((1, 4 * HID), lambda i: (0, 0)),
        ],
        out_specs=pl.BlockSpec((DAYS, TU, HID), lambda i: (0, i, 0)),
        out_shape=jax.ShapeDtypeStruct((DAYS, U, HID), f32),
    )(gx, onehot, ktp, md, zyw, wa1, wk, ab, wih, whh, lb)

    hflat = ys.reshape(U, DAYS * HID)
    gw = _pad2(gout_W, DAYS * HID, NLANE)
    xw = pl.pallas_call(
        _headmm,
        grid=(grid,),
        in_specs=[pl.BlockSpec((TU, DAYS * HID), lambda i: (i, 0)),
                  pl.BlockSpec((DAYS * HID, NLANE), lambda i: (0, 0))],
        out_specs=pl.BlockSpec((TU, NLANE), lambda i: (i, 0)),
        out_shape=jax.ShapeDtypeStruct((U, NLANE), f32),
    )(hflat, gw)[:, :OUT]

    xws = dinv[:, None] * xw
    g2 = dinv[:, None] * jax.ops.segment_sum(xws[src], dst, num_segments=U) \
        + slf[:, None] * xw

    bvec = jnp.full((1, NLANE), -1e30, f32).at[0, :OUT].set(gout_b)
    g2p = _pad2(g2, U, NLANE)
    probs = pl.pallas_call(
        _softmax,
        grid=(grid,),
        in_specs=[pl.BlockSpec((TU, NLANE), lambda i: (i, 0)),
                  pl.BlockSpec((1, NLANE), lambda i: (0, 0))],
        out_specs=pl.BlockSpec((TU, NLANE), lambda i: (i, 0)),
        out_shape=jax.ShapeDtypeStruct((U, NLANE), f32),
    )(g2p, bvec)[:, :OUT]

    micro = probs[None]
    macro = qtk[:C, :nh][None]
    return (micro, macro)


# speedup vs baseline: 57.6936x; 4.0277x over previous
import functools

import jax
import jax.numpy as jnp
from jax import lax
from jax.experimental import pallas as pl
from jax.experimental.pallas import tpu as pltpu
from jax.experimental.pallas import tpu_sc as plsc

HID = 128
DAYS = 10
NLANE = 128
TU = 1000

SC_NC = 2
SC_NS = 16
SC_NW = SC_NC * SC_NS
SC_W = 16


def _make_scprop(U, E):
    epw = E // SC_NW
    ch = epw
    while ch > 2000 or epw % ch != 0:
        ch = ch // 2 if ch % 2 == 0 else 1
    nchunk = epw // ch
    nrow = SC_NW * nchunk
    mesh = plsc.VectorSubcoreMesh(core_axis_name="c", subcore_axis_name="s")

    @functools.partial(
        pl.kernel, mesh=mesh,
        compiler_params=pltpu.CompilerParams(use_tc_tiling_on_sc=False),
        out_type=jax.ShapeDtypeStruct((SC_NC, U, SC_W), jnp.float32),
        scratch_types=[
            pltpu.VMEM((ch,), jnp.int32),
            pltpu.VMEM((ch,), jnp.int32),
            pltpu.VMEM((ch, SC_W), jnp.float32),
            pltpu.VMEM_SHARED((U, SC_W), jnp.float32),
            pltpu.SemaphoreType.DMA,
        ],
    )
    def scprop(xs_hbm, src_hbm, dst_hbm, zero_hbm, out_hbm,
               idx_v, dst_v, rows_v, acc_sh, sem):
        cid = lax.axis_index("c")
        sid = lax.axis_index("s")
        wid = sid * SC_NC + cid

        @pl.when(sid == 0)
        def _():
            pltpu.sync_copy(zero_hbm, acc_sh)

        plsc.subcore_barrier()
        for i in range(nchunk):
            row = wid * nchunk + i
            pltpu.sync_copy(src_hbm.at[row], idx_v)
            pltpu.async_copy(xs_hbm.at[idx_v], rows_v, sem).wait()
            pltpu.sync_copy(dst_hbm.at[row], dst_v)
            pltpu.sync_copy(rows_v, acc_sh.at[dst_v], add=True)
        plsc.subcore_barrier()
        for cc in range(SC_NC):
            @pl.when((sid == 0) & (cid == cc))
            def _():
                pltpu.sync_copy(acc_sh, out_hbm.at[cc])

    def call(xs, srcv, dstv, zero16):
        return scprop(xs, srcv.reshape(nrow, ch), dstv.reshape(nrow, ch),
                      zero16)

    return call


def _pad2(x, r, c):
    return jnp.zeros((r, c), jnp.float32).at[: x.shape[0], : x.shape[1]].set(x)


def _cstage(gc_ref, gm_ref, md_ref, nd_ref, wa2z_ref, wa2y_ref, ab_ref,
            a1_ref, a2_ref, whh_ref, mb_ref, sel_ref, zyw_ref, qtk_ref):
    gc = gc_ref[...]
    gm = gm_ref[...]
    wa2z = wa2z_ref[...]
    wa2y = wa2y_ref[...]
    ab = ab_ref[...]
    a1 = a1_ref[...]
    a2 = a2_ref[...]
    whh = whh_ref[...]
    mb = mb_ref[...]
    h = jnp.zeros((NLANE, NLANE), jnp.float32)
    c = jnp.zeros((NLANE, NLANE), jnp.float32)
    f32 = jnp.float32
    for d in range(DAYS):
        zt = jnp.dot(gc, md_ref[d], preferred_element_type=f32)
        yt = jnp.dot(gm, nd_ref[d], preferred_element_type=f32)
        zyw_ref[d] = (jnp.dot(zt, wa2z, preferred_element_type=f32)
                      + jnp.dot(yt, wa2y, preferred_element_type=f32) + ab)
        qz = jax.nn.relu(zt)
        qy = jax.nn.relu(yt)
        g = (jnp.dot(qz, a1, preferred_element_type=f32)
             + jnp.dot(qy, a2, preferred_element_type=f32)
             + jnp.dot(h, whh, preferred_element_type=f32) + mb)
        gi = jax.nn.sigmoid(jnp.dot(g, sel_ref[0], preferred_element_type=f32))
        gf = jax.nn.sigmoid(jnp.dot(g, sel_ref[1], preferred_element_type=f32))
        gg = jnp.tanh(jnp.dot(g, sel_ref[2], preferred_element_type=f32))
        go = jax.nn.sigmoid(jnp.dot(g, sel_ref[3], preferred_element_type=f32))
        c = gf * c + gi * gg
        h = go * jnp.tanh(c)
    qtk_ref[...] = h


def _nodestage(gx_ref, oh_ref, kt_ref, md_ref, zyw_ref, wa1_ref, wk_ref,
               ab_ref, wih_ref, whh_ref, lb_ref, ys_ref):
    f32 = jnp.float32
    gx = gx_ref[...]
    oh = oh_ref[...]
    wa1 = wa1_ref[...]
    wih = wih_ref[...]
    whh = whh_ref[...]
    lb = lb_ref[...]
    kterm = jnp.dot(kt_ref[...], wk_ref[...], preferred_element_type=f32) + ab_ref[...]
    h = jnp.zeros((TU, HID), jnp.float32)
    c = jnp.zeros((TU, HID), jnp.float32)
    for d in range(DAYS):
        ht = jnp.dot(gx, md_ref[d], preferred_element_type=f32)
        hr = jax.nn.relu(ht)
        hv1 = jax.nn.relu(jnp.dot(hr, wa1, preferred_element_type=f32)
                          + jnp.dot(oh, zyw_ref[d], preferred_element_type=f32)) * hr
        hv2 = jax.nn.relu(jnp.dot(hv1, wa1, preferred_element_type=f32) + kterm) * hv1
        g = (jnp.dot(hv2, wih, preferred_element_type=f32)
             + jnp.dot(h, whh, preferred_element_type=f32) + lb)
        gi = jax.nn.sigmoid(g[:, 0:HID])
        gf = jax.nn.sigmoid(g[:, HID:2 * HID])
        gg = jnp.tanh(g[:, 2 * HID:3 * HID])
        go = jax.nn.sigmoid(g[:, 3 * HID:4 * HID])
        c = gf * c + gi * gg
        h = go * jnp.tanh(c)
        ys_ref[d] = jax.nn.relu(h)


def _headmm(x_ref, w_ref, o_ref):
    o_ref[...] = jnp.dot(x_ref[...], w_ref[...], preferred_element_type=jnp.float32)


def _softmax(x_ref, b_ref, o_ref):
    y = x_ref[...] + b_ref[...]
    m = jnp.max(y, axis=1, keepdims=True)
    e = jnp.exp(y - m)
    o_ref[...] = e / jnp.sum(e, axis=1, keepdims=True)


def kernel(micro_dynamic, micro_edge_index, micro_node_degrees, com, com_edges,
           com_weights, macro_dynamic, macro_static, k_in, gcn_W, gcn_b,
           macro_gcn_W, macro_gcn_b, attn_W, attn_b, lstm_Wih, lstm_Whh,
           lstm_bih, lstm_bhh, mlstm_Wih, mlstm_Whh, mlstm_bih, mlstm_bhh,
           gout_W, gout_b):
    f32 = jnp.float32
    U = micro_dynamic.shape[0]
    C = macro_dynamic.shape[0]
    OUT = gout_W.shape[1]
    src, dst = micro_edge_index[0], micro_edge_index[1]

    deg = jax.ops.segment_sum(jnp.ones_like(src, f32), dst, num_segments=U) + 1.0
    dinv = jnp.where(deg > 0, deg ** -0.5, 0.0)
    slf = dinv * dinv
    bx = micro_dynamic[:, 0, :]
    xall = jnp.concatenate([bx, micro_node_degrees[:, None]], 1)
    E = src.shape[0]
    scprop = _make_scprop(U, E)
    zero16 = jnp.zeros((U, SC_W), f32)
    xs = _pad2(dinv[:, None] * xall, U, SC_W)
    agg = scprop(xs, src, dst, zero16).sum(0)[:, :xall.shape[1]]
    G = dinv[:, None] * agg + slf[:, None] * xall

    counts = jnp.maximum(jnp.bincount(com, length=C), 1).astype(f32)
    Gc = jax.ops.segment_sum(G, com, num_segments=C) / counts[:, None]

    msrc, mdst = com_edges[0], com_edges[1]
    degm = jax.ops.segment_sum(com_weights, mdst, num_segments=C) + 1.0
    dinvm = jnp.where(degm > 0, degm ** -0.5, 0.0)
    normm = dinvm[msrc] * com_weights * dinvm[mdst]
    slfm = dinvm * dinvm
    mx = jnp.concatenate([macro_dynamic[:, 0, :, 0], macro_dynamic[:, 0, :, 1],
                          macro_static[:, None]], 1)
    Gm = jax.ops.segment_sum(normm[:, None] * mx[msrc], mdst, num_segments=C) \
        + slfm[:, None] * mx

    eye = jnp.eye(NLANE, dtype=f32)
    md = jnp.stack([jnp.outer(eye[d], gcn_W[0]) + jnp.outer(eye[10], gcn_W[1])
                    + jnp.outer(eye[24], gcn_b) for d in range(DAYS)])
    nd = jnp.stack([jnp.outer(eye[d], macro_gcn_W[0])
                    + jnp.outer(eye[10 + d], macro_gcn_W[1])
                    + jnp.outer(eye[20], macro_gcn_W[2])
                    + jnp.outer(eye[24], macro_gcn_b) for d in range(DAYS)])
    wa1 = attn_W[:HID]
    wa2z = attn_W[HID:2 * HID]
    wa2y = attn_W[2 * HID:3 * HID]
    wk = attn_W[HID:].reshape(32, 8, HID).sum(0)
    ab = attn_b[None, :]
    mwih_t = mlstm_Wih.T
    a1 = _pad2(mwih_t[:HID], NLANE, NLANE)
    a2 = _pad2(mwih_t[HID:], NLANE, NLANE)
    mwhh = _pad2(mlstm_Whh.T, NLANE, NLANE)
    mb = _pad2((mlstm_bih + mlstm_bhh)[None, :], 1, NLANE)
    nh = mlstm_Whh.shape[0] // 4
    sel = jnp.zeros((4, NLANE, NLANE), f32)
    for j in range(4):
        for cix in range(nh):
            sel = sel.at[j, j * nh + cix, cix].set(1.0)
    wih = lstm_Wih.T
    whh = lstm_Whh.T
    lb = (lstm_bih + lstm_bhh)[None, :]

    gc_p = _pad2(Gc, NLANE, NLANE).at[:, 24].set(1.0)
    gm_p = _pad2(Gm, NLANE, NLANE).at[:, 24].set(1.0)

    zyw, qtk = pl.pallas_call(
        _cstage,
        out_shape=[jax.ShapeDtypeStruct((DAYS, NLANE, NLANE), f32),
                   jax.ShapeDtypeStruct((NLANE, NLANE), f32)],
    )(gc_p, gm_p, md, nd, wa2z, wa2y, ab, a1, a2, mwhh, mb, sel)

    gx = _pad2(G, U, NLANE).at[:, 24].set(1.0)
    onehot = (com[:, None] == jnp.arange(NLANE, dtype=com.dtype)[None, :]
              ).astype(f32)
    ktp = k_in.T

    grid = U // TU
    ys = pl.pallas_call(
        _nodestage,
        grid=(grid,),
        in_specs=[
            pl.BlockSpec((TU, NLANE), lambda i: (i, 0)),
            pl.BlockSpec((TU, NLANE), lambda i: (i, 0)),
            pl.BlockSpec((TU, 8), lambda i: (i, 0)),
            pl.BlockSpec((DAYS, NLANE, NLANE), lambda i: (0, 0, 0)),
            pl.BlockSpec((DAYS, NLANE, NLANE), lambda i: (0, 0, 0)),
            pl.BlockSpec((HID, HID), lambda i: (0, 0)),
            pl.BlockSpec((8, HID), lambda i: (0, 0)),
            pl.BlockSpec((1, HID), lambda i: (0, 0)),
            pl.BlockSpec((HID, 4 * HID), lambda i: (0, 0)),
            pl.BlockSpec((HID, 4 * HID), lambda i: (0, 0)),
            pl.BlockSpec((1, 4 * HID), lambda i: (0, 0)),
        ],
        out_specs=pl.BlockSpec((DAYS, TU, HID), lambda i: (0, i, 0)),
        out_shape=jax.ShapeDtypeStruct((DAYS, U, HID), f32),
    )(gx, onehot, ktp, md, zyw, wa1, wk, ab, wih, whh, lb)

    hflat = ys.reshape(U, DAYS * HID)
    gw = _pad2(gout_W, DAYS * HID, NLANE)
    xw = pl.pallas_call(
        _headmm,
        grid=(grid,),
        in_specs=[pl.BlockSpec((TU, DAYS * HID), lambda i: (i, 0)),
                  pl.BlockSpec((DAYS * HID, NLANE), lambda i: (0, 0))],
        out_specs=pl.BlockSpec((TU, NLANE), lambda i: (i, 0)),
        out_shape=jax.ShapeDtypeStruct((U, NLANE), f32),
    )(hflat, gw)[:, :OUT]

    xws = _pad2(dinv[:, None] * xw, U, SC_W)
    agg2 = scprop(xws, src, dst, zero16).sum(0)[:, :OUT]
    g2 = dinv[:, None] * agg2 + slf[:, None] * xw

    bvec = jnp.full((1, NLANE), -1e30, f32).at[0, :OUT].set(gout_b)
    g2p = _pad2(g2, U, NLANE)
    probs = pl.pallas_call(
        _softmax,
        grid=(grid,),
        in_specs=[pl.BlockSpec((TU, NLANE), lambda i: (i, 0)),
                  pl.BlockSpec((1, NLANE), lambda i: (0, 0))],
        out_specs=pl.BlockSpec((TU, NLANE), lambda i: (i, 0)),
        out_shape=jax.ShapeDtypeStruct((U, NLANE), f32),
    )(g2p, bvec)[:, :OUT]

    micro = probs[None]
    macro = qtk[:C, :nh][None]
    return (micro, macro)
